# baseline (speedup 1.0000x reference)
import functools

import jax
import jax.numpy as jnp
from jax import lax
from jax.experimental import pallas as pl
from jax.experimental.pallas import tpu as pltpu
from jax.experimental.pallas import tpu_sc as plsc

NC = 2
NS = 16
NW = NC * NS

CHUNK = 320
SUBS = ((0, 128), (128, 128), (256, 64))


def _sc_gather(B, VR, D):
    b_per_w = B // NW
    n_chunks = b_per_w // CHUNK
    assert b_per_w * NW == B and n_chunks * CHUNK == b_per_w

    mesh = plsc.VectorSubcoreMesh(
        core_axis_name="c", subcore_axis_name="s", num_cores=NC, num_subcores=NS
    )

    @functools.partial(
        pl.kernel,
        mesh=mesh,
        out_type=jax.ShapeDtypeStruct((B, D), jnp.float32),
        scratch_types=[
            pltpu.VMEM((2, CHUNK), jnp.int32),
            pltpu.VMEM((2, CHUNK, D), jnp.float32),
            pltpu.SemaphoreType.DMA,
            pltpu.SemaphoreType.DMA,
            pltpu.SemaphoreType.DMA,
            pltpu.SemaphoreType.DMA,
        ],
        compiler_params=pltpu.CompilerParams(use_tc_tiling_on_sc=False),
    )
    def k(idx_hbm, tok_hbm, out_hbm, idx_v, buf_v, g0, g1, w0, w1):
        wid = lax.axis_index("s") * NC + lax.axis_index("c")
        base = wid * b_per_w
        gsem = [g0, g1]
        wsem = [w0, w1]
        gd = {}
        wd = {}

        def start(c):
            s = c % 2
            off = base + c * CHUNK
            pltpu.sync_copy(idx_hbm.at[pl.ds(off, CHUNK)], idx_v.at[s])
            gd[s] = [
                pltpu.async_copy(
                    tok_hbm.at[idx_v.at[s].at[pl.ds(o, n)]],
                    buf_v.at[s].at[pl.ds(o, n)],
                    gsem[s],
                )
                for (o, n) in SUBS
            ]

        start(0)
        for c in range(n_chunks):
            s = c % 2
            if c + 1 < n_chunks:
                if c >= 1:
                    wd[(c + 1) % 2].wait()
                start(c + 1)
            for cp in gd[s]:
                cp.wait()
            wd[s] = pltpu.async_copy(
                buf_v.at[s], out_hbm.at[pl.ds(base + c * CHUNK, CHUNK)], wsem[s]
            )
        wd[(n_chunks - 2) % 2].wait()
        wd[(n_chunks - 1) % 2].wait()

    return k


def _tc_fixup(Bt, T, D):
    TB = 8
    BB = 256

    def body(g_ref, idx_ref, pos_ref, out_ref):
        x = g_ref[...]
        ti = pl.program_id(0)
        idxb = idx_ref[pl.ds(ti * TB, TB), :]
        par = (idxb & 1) == 1
        for t in range(TB):
            xt = jnp.transpose(x[:, t, :], (1, 0))
            sel = jnp.where(par[t:t + 1, :], xt[D:], xt[:D])
            out_ref[t] = sel + pos_ref[t][:, None]

    return pl.pallas_call(
        body,
        grid=(T // TB, Bt // BB),
        in_specs=[
            pl.BlockSpec((BB, TB, 2 * D), lambda ti, bi: (bi, ti, 0)),
            pl.BlockSpec((T, BB), lambda ti, bi: (0, bi)),
            pl.BlockSpec((TB, D), lambda ti, bi: (ti, 0)),
        ],
        out_specs=pl.BlockSpec((TB, D, BB), lambda ti, bi: (ti, 0, bi)),
        out_shape=jax.ShapeDtypeStruct((T, D, Bt), jnp.float32),
    )


def kernel(inputs, token_table, pos_table):
    Bt, T = inputs.shape
    V, D = token_table.shape
    B = Bt * T
    idx32 = inputs.astype(jnp.int32)
    idx_half = jnp.reshape(idx32 >> 1, (B,))
    tbl = jnp.reshape(token_table, (V // 2, 2 * D))
    gathered = _sc_gather(B, V // 2, 2 * D)(idx_half, tbl)
    g3 = jnp.reshape(gathered, (Bt, T, 2 * D))
    out_t = _tc_fixup(Bt, T, D)(g3, jnp.transpose(idx32), pos_table)
    return jnp.transpose(out_t, (2, 0, 1))
